# baseline (speedup 1.0000x reference)
import functools

import jax
import jax.numpy as jnp
from jax import lax
from jax.experimental import pallas as pl
from jax.experimental.pallas import tpu as pltpu
from jax.experimental.pallas import tpu_sc as plsc

N, E, D, H = 2048, 8, 2048, 1024
TN = 128
TT = TN * E
GRID = N // TN
NEG = float(jnp.finfo(jnp.float32).min)
DN = (((1,), (0,)), ((), ()))
_VM = pltpu.CompilerParams(dimension_semantics=("arbitrary",),
                           vmem_limit_bytes=100 * 1024 * 1024)


def _mm(l, w):
    return lax.dot_general(l, w, DN, preferred_element_type=jnp.float32)


def _mm_relu_body(x_ref, w_ref, b_ref, o_ref):
    x = x_ref[...].astype(jnp.float32)
    o_ref[...] = jnp.maximum(_mm(x, w_ref[...]) + b_ref[...], 0.0)


def _mm_relu(x, w, b):
    m, k = x.shape
    h = w.shape[1]
    return pl.pallas_call(
        _mm_relu_body,
        grid=(m // TT,),
        in_specs=[pl.BlockSpec((TT, k), lambda i: (i, 0)),
                  pl.BlockSpec((k, h), lambda i: (0, 0)),
                  pl.BlockSpec((1, h), lambda i: (0, 0))],
        out_specs=pl.BlockSpec((TT, h), lambda i: (i, 0)),
        out_shape=jax.ShapeDtypeStruct((m, h), jnp.float32),
        compiler_params=_VM,
    )(x, w, b)


def _pool_body(fi_ref, h_ref, me_ref, mx_ref, hf_ref, acc):
    h3 = h_ref[...].reshape(TN, E, H)
    acc[...] = h3[:, 0, :]
    for e in range(1, E):
        acc[...] = acc[...] + h3[:, e, :]
    me_ref[...] = acc[...] * (1.0 / E)
    mx_ref[...] = h3.max(axis=1)
    onehot = (lax.broadcasted_iota(jnp.int32, (1, E, 1), 1) == fi_ref[0, 0]
              ).astype(jnp.float32)
    hf_ref[...] = (h3 * onehot).sum(axis=1)


def _pools(fi, h2):
    return pl.pallas_call(
        _pool_body,
        grid=(GRID,),
        in_specs=[pl.BlockSpec(memory_space=pltpu.SMEM),
                  pl.BlockSpec((TT, H), lambda i: (i, 0))],
        out_specs=[pl.BlockSpec((TN, H), lambda i: (i, 0))] * 3,
        out_shape=[jax.ShapeDtypeStruct((N, H), jnp.float32)] * 3,
        scratch_shapes=[pltpu.VMEM((TN, H), jnp.float32)],
        compiler_params=_VM,
    )(fi, h2)


def _scorer_body(h_ref, me_ref, mx_ref, hf_ref, w_ref, b_ref, o_ref):
    h3 = h_ref[...].reshape(TN, E, H)
    feats = jnp.concatenate([
        h3,
        jnp.broadcast_to(me_ref[...][:, None, :], (TN, E, H)),
        jnp.broadcast_to(mx_ref[...][:, None, :], (TN, E, H)),
        jnp.broadcast_to(hf_ref[...][:, None, :], (TN, E, H))], axis=-1)
    o_ref[...] = jnp.maximum(_mm(feats.reshape(TT, 4 * H), w_ref[...])
                             + b_ref[...], 0.0)


def _scorer(h2, me, mx, hf, ws1, bs1):
    return pl.pallas_call(
        _scorer_body,
        grid=(GRID,),
        in_specs=[pl.BlockSpec((TT, H), lambda i: (i, 0)),
                  pl.BlockSpec((TN, H), lambda i: (i, 0)),
                  pl.BlockSpec((TN, H), lambda i: (i, 0)),
                  pl.BlockSpec((TN, H), lambda i: (i, 0)),
                  pl.BlockSpec((4 * H, H), lambda i: (0, 0)),
                  pl.BlockSpec((1, H), lambda i: (0, 0))],
        out_specs=pl.BlockSpec((TT, H), lambda i: (i, 0)),
        out_shape=jax.ShapeDtypeStruct((N * E, H), jnp.float32),
        compiler_params=_VM,
    )(h2, me, mx, hf, ws1, bs1)


def _logits_body(s_ref, w_ref, b_ref, o_ref):
    o_ref[...] = (_mm(s_ref[...], w_ref[...]) + b_ref[0, 0]).reshape(TN, E)


def _logits(sh, ws2, bs2):
    return pl.pallas_call(
        _logits_body,
        grid=(GRID,),
        in_specs=[pl.BlockSpec((TT, H), lambda i: (i, 0)),
                  pl.BlockSpec((H, 1), lambda i: (0, 0)),
                  pl.BlockSpec((1, 1), lambda i: (0, 0))],
        out_specs=pl.BlockSpec((TN, E), lambda i: (i, 0)),
        out_shape=jax.ShapeDtypeStruct((N, E), jnp.float32),
        compiler_params=_VM,
    )(sh, ws2, bs2)


def _defer_body(hf_ref, me_ref, mx_ref, wa, wb, wc, b_ref, w2_ref, b2_ref,
                o_ref):
    dh = jnp.maximum(_mm(hf_ref[...], wa[...]) + _mm(me_ref[...], wb[...])
                     + _mm(mx_ref[...], wc[...]) + b_ref[...], 0.0)
    z = _mm(dh, w2_ref[...]) + b2_ref[0, 0]
    o_ref[...] = 1.0 / (1.0 + jnp.exp(-z))


def _defer(hf, me, mx, wa, wb, wc, bd1, wd2, bd2):
    dt = 1024
    return pl.pallas_call(
        _defer_body,
        grid=(N // dt,),
        in_specs=[pl.BlockSpec((dt, H), lambda i: (i, 0)),
                  pl.BlockSpec((dt, H), lambda i: (i, 0)),
                  pl.BlockSpec((dt, H), lambda i: (i, 0)),
                  pl.BlockSpec((H, H), lambda i: (0, 0)),
                  pl.BlockSpec((H, H), lambda i: (0, 0)),
                  pl.BlockSpec((H, H), lambda i: (0, 0)),
                  pl.BlockSpec((1, H), lambda i: (0, 0)),
                  pl.BlockSpec((H, 1), lambda i: (0, 0)),
                  pl.BlockSpec((1, 1), lambda i: (0, 0))],
        out_specs=pl.BlockSpec((dt, 1), lambda i: (i, 0)),
        out_shape=jax.ShapeDtypeStruct((N, 1), jnp.float32),
        compiler_params=_VM,
    )(hf, me, mx, wa, wb, wc, bd1, wd2, bd2)


_NC, _NS = 2, 16
_NW = _NC * _NS
_ROWS = N // _NW
_GROUPS = _ROWS // 16

@functools.lru_cache(maxsize=1)
def _route_kernel():
    mesh = plsc.VectorSubcoreMesh(core_axis_name="c", subcore_axis_name="s")
    return functools.partial(
        pl.kernel, mesh=mesh,
        out_type=jax.ShapeDtypeStruct((N * E,), jnp.float32),
        scratch_types=[
            pltpu.VMEM((_ROWS * E,), jnp.float32),
            pltpu.VMEM((_ROWS * E,), jnp.float32),
            pltpu.VMEM((16,), jnp.int32),
        ],
        compiler_params=pltpu.CompilerParams(needs_layout_passes=False),
    )(_route_body)


def _route_body(logits_hbm, fi_hbm, out_hbm, slab, oslab, fi_v):
    wid = lax.axis_index("s") * _NC + lax.axis_index("c")
    base = wid * (_ROWS * E)
    pltpu.sync_copy(logits_hbm.at[pl.ds(base, _ROWS * E)], slab)
    pltpu.sync_copy(fi_hbm, fi_v)
    fiv = fi_v[...]
    lane = lax.iota(jnp.int32, 16)
    neg = jnp.full((16,), NEG, jnp.float32)

    for g in range(_GROUPS):
        ls, allowed, idxs = [], [], []
        for e in range(E):
            idx = lane * E + (g * 16 * E + e)
            idxs.append(idx)
            ls.append(plsc.load_gather(slab, [idx]))
            allowed.append(fiv != e)
        m1 = neg
        for e in range(E):
            m1 = jnp.maximum(m1, jnp.where(allowed[e], ls[e], neg))
        i1 = jnp.full((16,), E, jnp.int32)
        for e in range(E - 1, -1, -1):
            hit = allowed[e] & (ls[e] == m1)
            i1 = jnp.where(hit, jnp.full((16,), e, jnp.int32), i1)
        m2 = neg
        for e in range(E):
            ok = allowed[e] & (i1 != e)
            m2 = jnp.maximum(m2, jnp.where(ok, ls[e], neg))
        i2 = jnp.full((16,), E, jnp.int32)
        for e in range(E - 1, -1, -1):
            hit = allowed[e] & (i1 != e) & (ls[e] == m2)
            i2 = jnp.where(hit, jnp.full((16,), e, jnp.int32), i2)
        den = jnp.zeros((16,), jnp.float32)
        ws = []
        for e in range(E):
            keep = (i1 == e) | (i2 == e)
            w = jnp.where(keep, jnp.exp(ls[e] - m1), 0.0)
            den = den + w
            ws.append(w)
        inv = 1.0 / den
        for e in range(E):
            plsc.store_scatter(oslab, [idxs[e]], ws[e] * inv)

    pltpu.sync_copy(oslab, out_hbm.at[pl.ds(base, _ROWS * E)])


def kernel(tokens, W1, b1, W2, b2, Ws1, bs1, Ws2, bs2, Wd1, bd1, Wd2, bd2,
           full_index):
    x = tokens.reshape(N * E, D).astype(jnp.bfloat16)
    Wd1a, Wd1b, Wd1c = Wd1[:H], Wd1[H:2 * H], Wd1[2 * H:]
    fi = jnp.asarray(full_index, jnp.int32)

    h1 = _mm_relu(x, W1, b1.reshape(1, H))
    h2 = _mm_relu(h1.astype(jnp.bfloat16), W2, b2.reshape(1, H))
    me, mx, hf = _pools(fi.reshape(1, 1), h2)
    sh = _scorer(h2, me, mx, hf, Ws1, bs1.reshape(1, H))
    logits = _logits(sh, Ws2, bs2.reshape(1, 1))
    defer_prob = _defer(hf, me, mx, Wd1a, Wd1b, Wd1c, bd1.reshape(1, H),
                        Wd2, bd2.reshape(1, 1))
    weights = _route_kernel()(logits.reshape(N * E),
                              jnp.full((16,), fi, jnp.int32))
    return weights.reshape(N, E), defer_prob


# speedup vs baseline: 1.2309x; 1.2309x over previous
import functools

import jax
import jax.numpy as jnp
from jax import lax
from jax.experimental import pallas as pl
from jax.experimental.pallas import tpu as pltpu
from jax.experimental.pallas import tpu_sc as plsc

N, E, D, H = 2048, 8, 2048, 1024
TN = 128
TT = TN * E
GRID = N // TN
NEG = float(jnp.finfo(jnp.float32).min)
DN = (((1,), (0,)), ((), ()))
_VM = pltpu.CompilerParams(dimension_semantics=("arbitrary",),
                           vmem_limit_bytes=100 * 1024 * 1024)


def _mm(l, w):
    return lax.dot_general(l, w, DN, preferred_element_type=jnp.float32)


def _mm_relu_body(x_ref, w_ref, b_ref, o_ref):
    x = x_ref[...].astype(jnp.float32)
    r = jnp.maximum(_mm(x, w_ref[...]) + b_ref[...], 0.0)
    o_ref[...] = r.astype(o_ref.dtype)


def _mm_relu(x, w, b, out_dtype=jnp.float32):
    m, k = x.shape
    h = w.shape[1]
    return pl.pallas_call(
        _mm_relu_body,
        grid=(m // TT,),
        in_specs=[pl.BlockSpec((TT, k), lambda i: (i, 0)),
                  pl.BlockSpec((k, h), lambda i: (0, 0)),
                  pl.BlockSpec((1, h), lambda i: (0, 0))],
        out_specs=pl.BlockSpec((TT, h), lambda i: (i, 0)),
        out_shape=jax.ShapeDtypeStruct((m, h), out_dtype),
        compiler_params=_VM,
    )(x, w, b)


def _pool_body(fi_ref, h_ref, me_ref, mx_ref, hf_ref, acc):
    h3 = h_ref[...].reshape(TN, E, H)
    acc[...] = h3[:, 0, :]
    for e in range(1, E):
        acc[...] = acc[...] + h3[:, e, :]
    me_ref[...] = acc[...] * (1.0 / E)
    mx_ref[...] = h3.max(axis=1)
    onehot = (lax.broadcasted_iota(jnp.int32, (1, E, 1), 1) == fi_ref[0, 0]
              ).astype(jnp.float32)
    hf_ref[...] = (h3 * onehot).sum(axis=1)


def _pools(fi, h2):
    return pl.pallas_call(
        _pool_body,
        grid=(GRID,),
        in_specs=[pl.BlockSpec(memory_space=pltpu.SMEM),
                  pl.BlockSpec((TT, H), lambda i: (i, 0))],
        out_specs=[pl.BlockSpec((TN, H), lambda i: (i, 0))] * 3,
        out_shape=[jax.ShapeDtypeStruct((N, H), jnp.float32)] * 3,
        scratch_shapes=[pltpu.VMEM((TN, H), jnp.float32)],
        compiler_params=_VM,
    )(fi, h2)


def _scorer_body(h_ref, me_ref, mx_ref, hf_ref, w_ref, b_ref, o_ref):
    h3 = h_ref[...].reshape(TN, E, H)
    feats = jnp.concatenate([
        h3,
        jnp.broadcast_to(me_ref[...][:, None, :], (TN, E, H)),
        jnp.broadcast_to(mx_ref[...][:, None, :], (TN, E, H)),
        jnp.broadcast_to(hf_ref[...][:, None, :], (TN, E, H))], axis=-1)
    sh = jnp.maximum(_mm(feats.reshape(TT, 4 * H), w_ref[...])
                     + b_ref[...], 0.0)
    o_ref[...] = sh.astype(jnp.bfloat16)


def _scorer(h2, me, mx, hf, ws1, bs1):
    return pl.pallas_call(
        _scorer_body,
        grid=(GRID,),
        in_specs=[pl.BlockSpec((TT, H), lambda i: (i, 0)),
                  pl.BlockSpec((TN, H), lambda i: (i, 0)),
                  pl.BlockSpec((TN, H), lambda i: (i, 0)),
                  pl.BlockSpec((TN, H), lambda i: (i, 0)),
                  pl.BlockSpec((4 * H, H), lambda i: (0, 0)),
                  pl.BlockSpec((1, H), lambda i: (0, 0))],
        out_specs=pl.BlockSpec((TT, H), lambda i: (i, 0)),
        out_shape=jax.ShapeDtypeStruct((N * E, H), jnp.bfloat16),
        compiler_params=_VM,
    )(h2, me, mx, hf, ws1, bs1)


def _logits_body(s_ref, w_ref, b_ref, o_ref):
    s = s_ref[...].astype(jnp.float32)
    o_ref[...] = (_mm(s, w_ref[...]) + b_ref[0, 0]).reshape(TN, E)


def _logits(sh, ws2, bs2):
    return pl.pallas_call(
        _logits_body,
        grid=(GRID,),
        in_specs=[pl.BlockSpec((TT, H), lambda i: (i, 0)),
                  pl.BlockSpec((H, 1), lambda i: (0, 0)),
                  pl.BlockSpec((1, 1), lambda i: (0, 0))],
        out_specs=pl.BlockSpec((TN, E), lambda i: (i, 0)),
        out_shape=jax.ShapeDtypeStruct((N, E), jnp.float32),
        compiler_params=_VM,
    )(sh, ws2, bs2)


def _defer_body(hf_ref, me_ref, mx_ref, wa, wb, wc, b_ref, w2_ref, b2_ref,
                o_ref):
    dh = jnp.maximum(_mm(hf_ref[...], wa[...]) + _mm(me_ref[...], wb[...])
                     + _mm(mx_ref[...], wc[...]) + b_ref[...], 0.0)
    z = _mm(dh, w2_ref[...]) + b2_ref[0, 0]
    o_ref[...] = 1.0 / (1.0 + jnp.exp(-z))


def _defer(hf, me, mx, wa, wb, wc, bd1, wd2, bd2):
    dt = 1024
    return pl.pallas_call(
        _defer_body,
        grid=(N // dt,),
        in_specs=[pl.BlockSpec((dt, H), lambda i: (i, 0)),
                  pl.BlockSpec((dt, H), lambda i: (i, 0)),
                  pl.BlockSpec((dt, H), lambda i: (i, 0)),
                  pl.BlockSpec((H, H), lambda i: (0, 0)),
                  pl.BlockSpec((H, H), lambda i: (0, 0)),
                  pl.BlockSpec((H, H), lambda i: (0, 0)),
                  pl.BlockSpec((1, H), lambda i: (0, 0)),
                  pl.BlockSpec((H, 1), lambda i: (0, 0)),
                  pl.BlockSpec((1, 1), lambda i: (0, 0))],
        out_specs=pl.BlockSpec((dt, 1), lambda i: (i, 0)),
        out_shape=jax.ShapeDtypeStruct((N, 1), jnp.float32),
        compiler_params=_VM,
    )(hf, me, mx, wa, wb, wc, bd1, wd2, bd2)


_NC, _NS = 2, 16
_NW = _NC * _NS
_ROWS = N // _NW
_GROUPS = _ROWS // 16

@functools.lru_cache(maxsize=1)
def _route_kernel():
    mesh = plsc.VectorSubcoreMesh(core_axis_name="c", subcore_axis_name="s")
    return functools.partial(
        pl.kernel, mesh=mesh,
        out_type=jax.ShapeDtypeStruct((N * E,), jnp.float32),
        scratch_types=[
            pltpu.VMEM((_ROWS * E,), jnp.float32),
            pltpu.VMEM((_ROWS * E,), jnp.float32),
            pltpu.VMEM((16,), jnp.int32),
        ],
        compiler_params=pltpu.CompilerParams(needs_layout_passes=False),
    )(_route_body)


def _route_body(logits_hbm, fi_hbm, out_hbm, slab, oslab, fi_v):
    wid = lax.axis_index("s") * _NC + lax.axis_index("c")
    base = wid * (_ROWS * E)
    pltpu.sync_copy(logits_hbm.at[pl.ds(base, _ROWS * E)], slab)
    pltpu.sync_copy(fi_hbm, fi_v)
    fiv = fi_v[...]
    lane = lax.iota(jnp.int32, 16)
    neg = jnp.full((16,), NEG, jnp.float32)

    for g in range(_GROUPS):
        ls, allowed, idxs = [], [], []
        for e in range(E):
            idx = lane * E + (g * 16 * E + e)
            idxs.append(idx)
            ls.append(plsc.load_gather(slab, [idx]))
            allowed.append(fiv != e)
        m1 = neg
        for e in range(E):
            m1 = jnp.maximum(m1, jnp.where(allowed[e], ls[e], neg))
        i1 = jnp.full((16,), E, jnp.int32)
        for e in range(E - 1, -1, -1):
            hit = allowed[e] & (ls[e] == m1)
            i1 = jnp.where(hit, jnp.full((16,), e, jnp.int32), i1)
        m2 = neg
        for e in range(E):
            ok = allowed[e] & (i1 != e)
            m2 = jnp.maximum(m2, jnp.where(ok, ls[e], neg))
        i2 = jnp.full((16,), E, jnp.int32)
        for e in range(E - 1, -1, -1):
            hit = allowed[e] & (i1 != e) & (ls[e] == m2)
            i2 = jnp.where(hit, jnp.full((16,), e, jnp.int32), i2)
        den = jnp.zeros((16,), jnp.float32)
        ws = []
        for e in range(E):
            keep = (i1 == e) | (i2 == e)
            w = jnp.where(keep, jnp.exp(ls[e] - m1), 0.0)
            den = den + w
            ws.append(w)
        inv = 1.0 / den
        for e in range(E):
            plsc.store_scatter(oslab, [idxs[e]], ws[e] * inv)

    pltpu.sync_copy(oslab, out_hbm.at[pl.ds(base, _ROWS * E)])


def kernel(tokens, W1, b1, W2, b2, Ws1, bs1, Ws2, bs2, Wd1, bd1, Wd2, bd2,
           full_index):
    x = tokens.reshape(N * E, D)
    Wd1a, Wd1b, Wd1c = Wd1[:H], Wd1[H:2 * H], Wd1[2 * H:]
    fi = jnp.asarray(full_index, jnp.int32)

    h1 = _mm_relu(x, W1, b1.reshape(1, H), out_dtype=jnp.bfloat16)
    h2 = _mm_relu(h1, W2, b2.reshape(1, H))
    me, mx, hf = _pools(fi.reshape(1, 1), h2)
    sh = _scorer(h2, me, mx, hf, Ws1, bs1.reshape(1, H))
    logits = _logits(sh, Ws2, bs2.reshape(1, 1))
    defer_prob = _defer(hf, me, mx, Wd1a, Wd1b, Wd1c, bd1.reshape(1, H),
                        Wd2, bd2.reshape(1, 1))
    weights = _route_kernel()(logits.reshape(N * E),
                              jnp.full((16,), fi, jnp.int32))
    return weights.reshape(N, E), defer_prob


# speedup vs baseline: 1.2691x; 1.0310x over previous
import functools

import jax
import jax.numpy as jnp
from jax import lax
from jax.experimental import pallas as pl
from jax.experimental.pallas import tpu as pltpu
from jax.experimental.pallas import tpu_sc as plsc

N, E, D, H = 2048, 8, 2048, 1024
TN = 128
TT = TN * E
GRID = N // TN
NEG = float(jnp.finfo(jnp.float32).min)
DN = (((1,), (0,)), ((), ()))
_VM = pltpu.CompilerParams(dimension_semantics=("arbitrary",),
                           vmem_limit_bytes=100 * 1024 * 1024)


def _mm(l, w):
    return lax.dot_general(l, w, DN, preferred_element_type=jnp.float32)


def _mm_relu_body(x_ref, w_ref, b_ref, o_ref):
    x = x_ref[...].astype(jnp.float32)
    r = jnp.maximum(_mm(x, w_ref[...]) + b_ref[...], 0.0)
    o_ref[...] = r.astype(o_ref.dtype)


def _mm_relu(x, w, b, out_dtype=jnp.float32):
    m, k = x.shape
    h = w.shape[1]
    return pl.pallas_call(
        _mm_relu_body,
        grid=(m // TT,),
        in_specs=[pl.BlockSpec((TT, k), lambda i: (i, 0)),
                  pl.BlockSpec((k, h), lambda i: (0, 0)),
                  pl.BlockSpec((1, h), lambda i: (0, 0))],
        out_specs=pl.BlockSpec((TT, h), lambda i: (i, 0)),
        out_shape=jax.ShapeDtypeStruct((m, h), out_dtype),
        compiler_params=_VM,
    )(x, w, b)


def _pool_body(fi_ref, h_ref, w_ref, b_ref, h2_ref, me_ref, mx_ref,
               hf_ref, acc):
    x = h_ref[...].astype(jnp.float32)
    h2 = jnp.maximum(_mm(x, w_ref[...]) + b_ref[...], 0.0)
    h2_ref[...] = h2
    h3 = h2.reshape(TN, E, H)
    acc[...] = h3[:, 0, :]
    for e in range(1, E):
        acc[...] = acc[...] + h3[:, e, :]
    me_ref[...] = acc[...] * (1.0 / E)
    mx_ref[...] = h3.max(axis=1)
    onehot = (lax.broadcasted_iota(jnp.int32, (1, E, 1), 1) == fi_ref[0, 0]
              ).astype(jnp.float32)
    hf_ref[...] = (h3 * onehot).sum(axis=1)


def _pools(fi, h1, w2, b2):
    return pl.pallas_call(
        _pool_body,
        grid=(GRID,),
        in_specs=[pl.BlockSpec(memory_space=pltpu.SMEM),
                  pl.BlockSpec((TT, H), lambda i: (i, 0)),
                  pl.BlockSpec((H, H), lambda i: (0, 0)),
                  pl.BlockSpec((1, H), lambda i: (0, 0))],
        out_specs=[pl.BlockSpec((TT, H), lambda i: (i, 0))]
        + [pl.BlockSpec((TN, H), lambda i: (i, 0))] * 3,
        out_shape=[jax.ShapeDtypeStruct((N * E, H), jnp.float32)]
        + [jax.ShapeDtypeStruct((N, H), jnp.float32)] * 3,
        scratch_shapes=[pltpu.VMEM((TN, H), jnp.float32)],
        compiler_params=_VM,
    )(fi, h1, w2, b2)


def _scorer_body(h_ref, me_ref, mx_ref, hf_ref, w_ref, b_ref, o_ref):
    h3 = h_ref[...].reshape(TN, E, H)
    feats = jnp.concatenate([
        h3,
        jnp.broadcast_to(me_ref[...][:, None, :], (TN, E, H)),
        jnp.broadcast_to(mx_ref[...][:, None, :], (TN, E, H)),
        jnp.broadcast_to(hf_ref[...][:, None, :], (TN, E, H))], axis=-1)
    sh = jnp.maximum(_mm(feats.reshape(TT, 4 * H), w_ref[...])
                     + b_ref[...], 0.0)
    o_ref[...] = sh.astype(jnp.bfloat16)


def _scorer(h2, me, mx, hf, ws1, bs1):
    return pl.pallas_call(
        _scorer_body,
        grid=(GRID,),
        in_specs=[pl.BlockSpec((TT, H), lambda i: (i, 0)),
                  pl.BlockSpec((TN, H), lambda i: (i, 0)),
                  pl.BlockSpec((TN, H), lambda i: (i, 0)),
                  pl.BlockSpec((TN, H), lambda i: (i, 0)),
                  pl.BlockSpec((4 * H, H), lambda i: (0, 0)),
                  pl.BlockSpec((1, H), lambda i: (0, 0))],
        out_specs=pl.BlockSpec((TT, H), lambda i: (i, 0)),
        out_shape=jax.ShapeDtypeStruct((N * E, H), jnp.bfloat16),
        compiler_params=_VM,
    )(h2, me, mx, hf, ws1, bs1)


def _logits_body(s_ref, w_ref, b_ref, o_ref):
    s = s_ref[...].astype(jnp.float32)
    o_ref[...] = (_mm(s, w_ref[...]) + b_ref[0, 0]).reshape(TN, E)


def _logits(sh, ws2, bs2):
    return pl.pallas_call(
        _logits_body,
        grid=(GRID,),
        in_specs=[pl.BlockSpec((TT, H), lambda i: (i, 0)),
                  pl.BlockSpec((H, 1), lambda i: (0, 0)),
                  pl.BlockSpec((1, 1), lambda i: (0, 0))],
        out_specs=pl.BlockSpec((TN, E), lambda i: (i, 0)),
        out_shape=jax.ShapeDtypeStruct((N, E), jnp.float32),
        compiler_params=_VM,
    )(sh, ws2, bs2)


def _defer_body(hf_ref, me_ref, mx_ref, wa, wb, wc, b_ref, w2_ref, b2_ref,
                o_ref):
    dh = jnp.maximum(_mm(hf_ref[...], wa[...]) + _mm(me_ref[...], wb[...])
                     + _mm(mx_ref[...], wc[...]) + b_ref[...], 0.0)
    z = _mm(dh, w2_ref[...]) + b2_ref[0, 0]
    o_ref[...] = 1.0 / (1.0 + jnp.exp(-z))


def _defer(hf, me, mx, wa, wb, wc, bd1, wd2, bd2):
    dt = 1024
    return pl.pallas_call(
        _defer_body,
        grid=(N // dt,),
        in_specs=[pl.BlockSpec((dt, H), lambda i: (i, 0)),
                  pl.BlockSpec((dt, H), lambda i: (i, 0)),
                  pl.BlockSpec((dt, H), lambda i: (i, 0)),
                  pl.BlockSpec((H, H), lambda i: (0, 0)),
                  pl.BlockSpec((H, H), lambda i: (0, 0)),
                  pl.BlockSpec((H, H), lambda i: (0, 0)),
                  pl.BlockSpec((1, H), lambda i: (0, 0)),
                  pl.BlockSpec((H, 1), lambda i: (0, 0)),
                  pl.BlockSpec((1, 1), lambda i: (0, 0))],
        out_specs=pl.BlockSpec((dt, 1), lambda i: (i, 0)),
        out_shape=jax.ShapeDtypeStruct((N, 1), jnp.float32),
        compiler_params=_VM,
    )(hf, me, mx, wa, wb, wc, bd1, wd2, bd2)


_NC, _NS = 2, 16
_NW = _NC * _NS
_ROWS = N // _NW
_GROUPS = _ROWS // 16

@functools.lru_cache(maxsize=1)
def _route_kernel():
    mesh = plsc.VectorSubcoreMesh(core_axis_name="c", subcore_axis_name="s")
    return functools.partial(
        pl.kernel, mesh=mesh,
        out_type=jax.ShapeDtypeStruct((N * E,), jnp.float32),
        scratch_types=[
            pltpu.VMEM((_ROWS * E,), jnp.float32),
            pltpu.VMEM((_ROWS * E,), jnp.float32),
            pltpu.VMEM((16,), jnp.int32),
        ],
        compiler_params=pltpu.CompilerParams(needs_layout_passes=False),
    )(_route_body)


def _route_body(logits_hbm, fi_hbm, out_hbm, slab, oslab, fi_v):
    wid = lax.axis_index("s") * _NC + lax.axis_index("c")
    base = wid * (_ROWS * E)
    pltpu.sync_copy(logits_hbm.at[pl.ds(base, _ROWS * E)], slab)
    pltpu.sync_copy(fi_hbm, fi_v)
    fiv = fi_v[...]
    lane = lax.iota(jnp.int32, 16)
    neg = jnp.full((16,), NEG, jnp.float32)

    for g in range(_GROUPS):
        ls, allowed, idxs = [], [], []
        for e in range(E):
            idx = lane * E + (g * 16 * E + e)
            idxs.append(idx)
            ls.append(plsc.load_gather(slab, [idx]))
            allowed.append(fiv != e)
        m1 = neg
        for e in range(E):
            m1 = jnp.maximum(m1, jnp.where(allowed[e], ls[e], neg))
        i1 = jnp.full((16,), E, jnp.int32)
        for e in range(E - 1, -1, -1):
            hit = allowed[e] & (ls[e] == m1)
            i1 = jnp.where(hit, jnp.full((16,), e, jnp.int32), i1)
        m2 = neg
        for e in range(E):
            ok = allowed[e] & (i1 != e)
            m2 = jnp.maximum(m2, jnp.where(ok, ls[e], neg))
        i2 = jnp.full((16,), E, jnp.int32)
        for e in range(E - 1, -1, -1):
            hit = allowed[e] & (i1 != e) & (ls[e] == m2)
            i2 = jnp.where(hit, jnp.full((16,), e, jnp.int32), i2)
        den = jnp.zeros((16,), jnp.float32)
        ws = []
        for e in range(E):
            keep = (i1 == e) | (i2 == e)
            w = jnp.where(keep, jnp.exp(ls[e] - m1), 0.0)
            den = den + w
            ws.append(w)
        inv = 1.0 / den
        for e in range(E):
            plsc.store_scatter(oslab, [idxs[e]], ws[e] * inv)

    pltpu.sync_copy(oslab, out_hbm.at[pl.ds(base, _ROWS * E)])


def kernel(tokens, W1, b1, W2, b2, Ws1, bs1, Ws2, bs2, Wd1, bd1, Wd2, bd2,
           full_index):
    x = tokens.reshape(N * E, D)
    Wd1a, Wd1b, Wd1c = Wd1[:H], Wd1[H:2 * H], Wd1[2 * H:]
    fi = jnp.asarray(full_index, jnp.int32)

    h1 = _mm_relu(x, W1, b1.reshape(1, H), out_dtype=jnp.bfloat16)
    h2, me, mx, hf = _pools(fi.reshape(1, 1), h1, W2, b2.reshape(1, H))
    sh = _scorer(h2, me, mx, hf, Ws1, bs1.reshape(1, H))
    logits = _logits(sh, Ws2, bs2.reshape(1, 1))
    defer_prob = _defer(hf, me, mx, Wd1a, Wd1b, Wd1c, bd1.reshape(1, H),
                        Wd2, bd2.reshape(1, 1))
    weights = _route_kernel()(logits.reshape(N * E),
                              jnp.full((16,), fi, jnp.int32))
    return weights.reshape(N, E), defer_prob


# speedup vs baseline: 1.3020x; 1.0259x over previous
import functools

import jax
import jax.numpy as jnp
from jax import lax
from jax.experimental import pallas as pl
from jax.experimental.pallas import tpu as pltpu
from jax.experimental.pallas import tpu_sc as plsc

N, E, D, H = 2048, 8, 2048, 1024
TN = 128
TT = TN * E
GRID = N // TN
NEG = float(jnp.finfo(jnp.float32).min)
DN = (((1,), (0,)), ((), ()))
_VM = pltpu.CompilerParams(dimension_semantics=("arbitrary",),
                           vmem_limit_bytes=100 * 1024 * 1024)


def _mm(l, w):
    return lax.dot_general(l, w, DN, preferred_element_type=jnp.float32)


def _mm_relu_body(x_ref, w_ref, b_ref, o_ref):
    x = x_ref[...].astype(jnp.float32)
    r = jnp.maximum(_mm(x, w_ref[...]) + b_ref[...], 0.0)
    o_ref[...] = r.astype(o_ref.dtype)


def _mm_relu(x, w, b, out_dtype=jnp.float32):
    m, k = x.shape
    h = w.shape[1]
    return pl.pallas_call(
        _mm_relu_body,
        grid=(m // TT,),
        in_specs=[pl.BlockSpec((TT, k), lambda i: (i, 0)),
                  pl.BlockSpec((k, h), lambda i: (0, 0)),
                  pl.BlockSpec((1, h), lambda i: (0, 0))],
        out_specs=pl.BlockSpec((TT, h), lambda i: (i, 0)),
        out_shape=jax.ShapeDtypeStruct((m, h), out_dtype),
        compiler_params=_VM,
    )(x, w, b)


def _pool_body(fi_ref, h_ref, w_ref, b_ref, h2_ref, me_ref, mx_ref,
               hf_ref, acc):
    x = h_ref[...].astype(jnp.float32)
    h2 = jnp.maximum(_mm(x, w_ref[...]) + b_ref[...], 0.0)
    h2_ref[...] = h2
    h3 = h2.reshape(TN, E, H)
    acc[...] = h3[:, 0, :]
    for e in range(1, E):
        acc[...] = acc[...] + h3[:, e, :]
    me_ref[...] = acc[...] * (1.0 / E)
    mx_ref[...] = h3.max(axis=1)
    onehot = (lax.broadcasted_iota(jnp.int32, (1, E, 1), 1) == fi_ref[0, 0]
              ).astype(jnp.float32)
    hf_ref[...] = (h3 * onehot).sum(axis=1)


def _pools(fi, h1, w2, b2):
    return pl.pallas_call(
        _pool_body,
        grid=(GRID,),
        in_specs=[pl.BlockSpec(memory_space=pltpu.SMEM),
                  pl.BlockSpec((TT, H), lambda i: (i, 0)),
                  pl.BlockSpec((H, H), lambda i: (0, 0)),
                  pl.BlockSpec((1, H), lambda i: (0, 0))],
        out_specs=[pl.BlockSpec((TT, H), lambda i: (i, 0))]
        + [pl.BlockSpec((TN, H), lambda i: (i, 0))] * 3,
        out_shape=[jax.ShapeDtypeStruct((N * E, H), jnp.float32)]
        + [jax.ShapeDtypeStruct((N, H), jnp.float32)] * 3,
        scratch_shapes=[pltpu.VMEM((TN, H), jnp.float32)],
        compiler_params=_VM,
    )(fi, h1, w2, b2)


def _scorer_body(h_ref, me_ref, mx_ref, hf_ref, w_ref, b_ref, w2_ref,
                 b2_ref, o_ref):
    h3 = h_ref[...].reshape(TN, E, H)
    feats = jnp.concatenate([
        h3,
        jnp.broadcast_to(me_ref[...][:, None, :], (TN, E, H)),
        jnp.broadcast_to(mx_ref[...][:, None, :], (TN, E, H)),
        jnp.broadcast_to(hf_ref[...][:, None, :], (TN, E, H))], axis=-1)
    sh = jnp.maximum(_mm(feats.reshape(TT, 4 * H), w_ref[...])
                     + b_ref[...], 0.0)
    sh = sh.astype(jnp.bfloat16).astype(jnp.float32)
    o_ref[...] = (_mm(sh, w2_ref[...]) + b2_ref[0, 0]).reshape(TN, E)


def _scorer(h2, me, mx, hf, ws1, bs1, ws2, bs2):
    return pl.pallas_call(
        _scorer_body,
        grid=(GRID,),
        in_specs=[pl.BlockSpec((TT, H), lambda i: (i, 0)),
                  pl.BlockSpec((TN, H), lambda i: (i, 0)),
                  pl.BlockSpec((TN, H), lambda i: (i, 0)),
                  pl.BlockSpec((TN, H), lambda i: (i, 0)),
                  pl.BlockSpec((4 * H, H), lambda i: (0, 0)),
                  pl.BlockSpec((1, H), lambda i: (0, 0)),
                  pl.BlockSpec((H, 1), lambda i: (0, 0)),
                  pl.BlockSpec((1, 1), lambda i: (0, 0))],
        out_specs=pl.BlockSpec((TN, E), lambda i: (i, 0)),
        out_shape=jax.ShapeDtypeStruct((N, E), jnp.float32),
        compiler_params=_VM,
    )(h2, me, mx, hf, ws1, bs1, ws2, bs2)


def _logits_body(s_ref, w_ref, b_ref, o_ref):
    s = s_ref[...].astype(jnp.float32)
    o_ref[...] = (_mm(s, w_ref[...]) + b_ref[0, 0]).reshape(TN, E)


def _logits(sh, ws2, bs2):
    return pl.pallas_call(
        _logits_body,
        grid=(GRID,),
        in_specs=[pl.BlockSpec((TT, H), lambda i: (i, 0)),
                  pl.BlockSpec((H, 1), lambda i: (0, 0)),
                  pl.BlockSpec((1, 1), lambda i: (0, 0))],
        out_specs=pl.BlockSpec((TN, E), lambda i: (i, 0)),
        out_shape=jax.ShapeDtypeStruct((N, E), jnp.float32),
        compiler_params=_VM,
    )(sh, ws2, bs2)


def _defer_body(hf_ref, me_ref, mx_ref, wa, wb, wc, b_ref, w2_ref, b2_ref,
                o_ref):
    dh = jnp.maximum(_mm(hf_ref[...], wa[...]) + _mm(me_ref[...], wb[...])
                     + _mm(mx_ref[...], wc[...]) + b_ref[...], 0.0)
    z = _mm(dh, w2_ref[...]) + b2_ref[0, 0]
    o_ref[...] = 1.0 / (1.0 + jnp.exp(-z))


def _defer(hf, me, mx, wa, wb, wc, bd1, wd2, bd2):
    dt = 1024
    return pl.pallas_call(
        _defer_body,
        grid=(N // dt,),
        in_specs=[pl.BlockSpec((dt, H), lambda i: (i, 0)),
                  pl.BlockSpec((dt, H), lambda i: (i, 0)),
                  pl.BlockSpec((dt, H), lambda i: (i, 0)),
                  pl.BlockSpec((H, H), lambda i: (0, 0)),
                  pl.BlockSpec((H, H), lambda i: (0, 0)),
                  pl.BlockSpec((H, H), lambda i: (0, 0)),
                  pl.BlockSpec((1, H), lambda i: (0, 0)),
                  pl.BlockSpec((H, 1), lambda i: (0, 0)),
                  pl.BlockSpec((1, 1), lambda i: (0, 0))],
        out_specs=pl.BlockSpec((dt, 1), lambda i: (i, 0)),
        out_shape=jax.ShapeDtypeStruct((N, 1), jnp.float32),
        compiler_params=_VM,
    )(hf, me, mx, wa, wb, wc, bd1, wd2, bd2)


_NC, _NS = 2, 16
_NW = _NC * _NS
_ROWS = N // _NW
_GROUPS = _ROWS // 16

@functools.lru_cache(maxsize=1)
def _route_kernel():
    mesh = plsc.VectorSubcoreMesh(core_axis_name="c", subcore_axis_name="s")
    return functools.partial(
        pl.kernel, mesh=mesh,
        out_type=jax.ShapeDtypeStruct((N * E,), jnp.float32),
        scratch_types=[
            pltpu.VMEM((_ROWS * E,), jnp.float32),
            pltpu.VMEM((_ROWS * E,), jnp.float32),
            pltpu.VMEM((16,), jnp.int32),
        ],
        compiler_params=pltpu.CompilerParams(needs_layout_passes=False),
    )(_route_body)


def _route_body(logits_hbm, fi_hbm, out_hbm, slab, oslab, fi_v):
    wid = lax.axis_index("s") * _NC + lax.axis_index("c")
    base = wid * (_ROWS * E)
    pltpu.sync_copy(logits_hbm.at[pl.ds(base, _ROWS * E)], slab)
    pltpu.sync_copy(fi_hbm, fi_v)
    fiv = fi_v[...]
    lane = lax.iota(jnp.int32, 16)
    neg = jnp.full((16,), NEG, jnp.float32)

    for g in range(_GROUPS):
        ls, allowed, idxs = [], [], []
        for e in range(E):
            idx = lane * E + (g * 16 * E + e)
            idxs.append(idx)
            ls.append(plsc.load_gather(slab, [idx]))
            allowed.append(fiv != e)
        m1 = neg
        for e in range(E):
            m1 = jnp.maximum(m1, jnp.where(allowed[e], ls[e], neg))
        i1 = jnp.full((16,), E, jnp.int32)
        for e in range(E - 1, -1, -1):
            hit = allowed[e] & (ls[e] == m1)
            i1 = jnp.where(hit, jnp.full((16,), e, jnp.int32), i1)
        m2 = neg
        for e in range(E):
            ok = allowed[e] & (i1 != e)
            m2 = jnp.maximum(m2, jnp.where(ok, ls[e], neg))
        i2 = jnp.full((16,), E, jnp.int32)
        for e in range(E - 1, -1, -1):
            hit = allowed[e] & (i1 != e) & (ls[e] == m2)
            i2 = jnp.where(hit, jnp.full((16,), e, jnp.int32), i2)
        den = jnp.zeros((16,), jnp.float32)
        ws = []
        for e in range(E):
            keep = (i1 == e) | (i2 == e)
            w = jnp.where(keep, jnp.exp(ls[e] - m1), 0.0)
            den = den + w
            ws.append(w)
        inv = 1.0 / den
        for e in range(E):
            plsc.store_scatter(oslab, [idxs[e]], ws[e] * inv)

    pltpu.sync_copy(oslab, out_hbm.at[pl.ds(base, _ROWS * E)])


def kernel(tokens, W1, b1, W2, b2, Ws1, bs1, Ws2, bs2, Wd1, bd1, Wd2, bd2,
           full_index):
    x = tokens.reshape(N * E, D)
    Wd1a, Wd1b, Wd1c = Wd1[:H], Wd1[H:2 * H], Wd1[2 * H:]
    fi = jnp.asarray(full_index, jnp.int32)

    h1 = _mm_relu(x, W1, b1.reshape(1, H), out_dtype=jnp.bfloat16)
    h2, me, mx, hf = _pools(fi.reshape(1, 1), h1, W2, b2.reshape(1, H))
    logits = _scorer(h2, me, mx, hf, Ws1, bs1.reshape(1, H), Ws2,
                     bs2.reshape(1, 1))
    defer_prob = _defer(hf, me, mx, Wd1a, Wd1b, Wd1c, bd1.reshape(1, H),
                        Wd2, bd2.reshape(1, 1))
    weights = _route_kernel()(logits.reshape(N * E),
                              jnp.full((16,), fi, jnp.int32))
    return weights.reshape(N, E), defer_prob
